# baseline (speedup 1.0000x reference)
import functools

import jax
import jax.numpy as jnp
from jax import lax
from jax.experimental import pallas as pl
from jax.experimental.pallas import tpu as pltpu
from jax.experimental.pallas import tpu_sc as plsc

B = 32
V = 25000
F = 50000
L = 16
NC, NS = 2, 16

CH = 1280
NFULL = F // CH
REM = F - NFULL * CH


def _body(vT, fT, oT, verts_v, faces_v, out_v, verts_sp,
          faces_spA, faces_spB, out_spA, out_spB):
    c = lax.axis_index("c")
    s = lax.axis_index("s")
    b0 = c * NS

    for h in range(NS):
        @pl.when(s == 0)
        def _():
            for k in range(3):
                pltpu.sync_copy(vT.at[k, b0 + h], verts_sp.at[k])

        plsc.subcore_barrier()

        @pl.when(s == h)
        def _():
            for k in range(3):
                pltpu.sync_copy(verts_sp.at[k], verts_v.at[k])

        plsc.subcore_barrier()

    def stage_faces(f_sp, f0, w):
        for j in range(3):
            @pl.when(s == 9 + j)
            def _():
                pltpu.sync_copy(
                    fT.at[j, pl.ds(b0, NS), pl.ds(f0, w)],
                    f_sp.at[j, :, pl.ds(0, w)],
                )

    def compute_chunk(f_sp, o_sp, w):
        nj = w // L
        pltpu.sync_copy(
            f_sp.at[:, s, pl.ds(0, w)], faces_v.at[:, pl.ds(0, w)]
        )

        @plsc.parallel_loop(0, nj, step=1, unroll=4)
        def _loop(i):
            for j in range(3):
                fj = faces_v[j, pl.ds(i * L, L)]
                for k in range(3):
                    row = jnp.full((L,), k, jnp.int32)
                    vals = plsc.load_gather(verts_v, [row, fj])
                    out_v[3 * j + k, pl.ds(i * L, L)] = vals

        pltpu.sync_copy(
            out_v.at[:, pl.ds(0, w)], o_sp.at[:, s, pl.ds(0, w)]
        )

    def flush_out(o_sp, f0, w):
        for r in range(9):
            @pl.when(s == r)
            def _():
                pltpu.sync_copy(
                    o_sp.at[r, :, pl.ds(0, w)],
                    oT.at[r, pl.ds(b0, NS), pl.ds(f0, w)],
                )

    stage_faces(faces_spA, 0, CH)
    for j in range(3):
        @pl.when(s == 12 + j)
        def _():
            pltpu.sync_copy(
                fT.at[j, pl.ds(b0, NS), pl.ds(CH, CH)],
                faces_spB.at[j, :, pl.ds(0, CH)],
            )
    plsc.subcore_barrier()

    def pair_step(q, carry):
        for p, (f_sp, o_sp) in enumerate(
            ((faces_spA, out_spA), (faces_spB, out_spB))
        ):
            ci = 2 * q + p
            f0 = pl.multiple_of(ci * CH, 128)
            compute_chunk(f_sp, o_sp, CH)
            plsc.subcore_barrier()
            flush_out(o_sp, f0, CH)

            @pl.when(ci + 2 < NFULL)
            def _():
                stage_faces(f_sp, pl.multiple_of(f0 + 2 * CH, 128), CH)

        return carry

    lax.fori_loop(0, NFULL // 2, pair_step, 0)

    if NFULL % 2:
        compute_chunk(faces_spA, out_spA, CH)
        plsc.subcore_barrier()
        flush_out(out_spA, (NFULL - 1) * CH, CH)
    if REM:
        f_sp, o_sp = (faces_spB, out_spB) if NFULL % 2 else (faces_spA, out_spA)
        stage_faces(f_sp, NFULL * CH, REM)
        plsc.subcore_barrier()
        compute_chunk(f_sp, o_sp, REM)
        plsc.subcore_barrier()
        flush_out(o_sp, NFULL * CH, REM)


@functools.partial(jax.jit, static_argnames=())
def kernel(vertices, faces):
    vT = vertices.transpose(2, 0, 1)
    fT = faces.transpose(2, 0, 1)
    mesh = plsc.VectorSubcoreMesh(
        core_axis_name="c", subcore_axis_name="s", num_cores=NC, num_subcores=NS
    )
    outT = pl.kernel(
        _body,
        out_type=jax.ShapeDtypeStruct((9, B, F), jnp.float32),
        mesh=mesh,
        compiler_params=pltpu.CompilerParams(
            needs_layout_passes=False, use_tc_tiling_on_sc=False
        ),
        scratch_types=[
            pltpu.VMEM((3, V), jnp.float32),
            pltpu.VMEM((3, CH), jnp.int32),
            pltpu.VMEM((9, CH), jnp.float32),
            pltpu.VMEM_SHARED((3, V), jnp.float32),
            pltpu.VMEM_SHARED((3, NS, CH), jnp.int32),
            pltpu.VMEM_SHARED((3, NS, CH), jnp.int32),
            pltpu.VMEM_SHARED((9, NS, CH), jnp.float32),
            pltpu.VMEM_SHARED((9, NS, CH), jnp.float32),
        ],
    )(vT, fT)
    return outT.transpose(1, 2, 0)
